# baseline (speedup 1.0000x reference)
import functools

import jax
import jax.numpy as jnp
from jax import lax
from jax.experimental import pallas as pl
from jax.experimental.pallas import tpu as pltpu
from jax.experimental.pallas import tpu_sc as plsc

N = 10000
E = 320000
D = 128
DH = D // 2

NC = 2
NS = 16
NW = NC * NS

CH = 128
KD = 80
KS = 160
E_PAD = NW * KD * CH
NP = 10240
RPT = NP // NS
DEGW = 16
NBUF = 8
LOOKAHEAD = 3
KH = KS // 2

_mesh = lambda: plsc.VectorSubcoreMesh(core_axis_name="c", subcore_axis_name="s")


@functools.partial(
    pl.kernel,
    out_type=jax.ShapeDtypeStruct((NC, NP, DEGW), jnp.float32),
    mesh=_mesh(),
    scratch_types=[
        pltpu.VMEM((KD, CH), jnp.int32),
        pltpu.VMEM((CH, DEGW), jnp.float32),
        pltpu.VMEM((CH, DEGW), jnp.float32),
        pltpu.VMEM_SHARED((NP, DEGW), jnp.float32),
    ],
    compiler_params=pltpu.CompilerParams(use_tc_tiling_on_sc=False),
)
def _deg_kernel(dst_hbm, out_hbm, idx_v, ones_v, zero_v, deg_sh):
    cid = lax.axis_index("c")
    sid = lax.axis_index("s")
    wid = cid * NS + sid
    one = jnp.full((16,), 1.0, jnp.float32)
    zero = jnp.zeros((16,), jnp.float32)

    def fill(i, _):
        ones_v[i, :] = one
        zero_v[i, :] = zero
        return _

    lax.fori_loop(0, CH, fill, None)

    def zrow(b, _):
        pltpu.sync_copy(zero_v, deg_sh.at[pl.ds(sid * RPT + b * CH, CH)])
        return _

    lax.fori_loop(0, RPT // CH, zrow, None)
    pltpu.sync_copy(dst_hbm.at[pl.ds(wid * KD, KD)], idx_v)
    plsc.subcore_barrier()

    def body(j, _):
        pltpu.sync_copy(ones_v, deg_sh.at[idx_v.at[j]], add=True)
        return _

    lax.fori_loop(0, KD, body, None)
    plsc.subcore_barrier()
    pltpu.sync_copy(deg_sh.at[pl.ds(sid * RPT, RPT)],
                    out_hbm.at[cid, pl.ds(sid * RPT, RPT)])


@functools.partial(
    pl.kernel,
    out_type=jax.ShapeDtypeStruct((NC, NP, DH), jnp.float32),
    mesh=_mesh(),
    scratch_types=[
        pltpu.VMEM((KH, CH), jnp.int32),
        pltpu.VMEM((KH, CH), jnp.int32),
        pltpu.VMEM((NBUF, CH, DH), jnp.float32),
        pltpu.VMEM_SHARED((NP, DH), jnp.float32),
        [pltpu.SemaphoreType.DMA] * NBUF,
        [pltpu.SemaphoreType.DMA] * NBUF,
    ],
    compiler_params=pltpu.CompilerParams(use_tc_tiling_on_sc=False),
)
def _scatter_kernel(y2_hbm, src_hbm, dst_hbm, out_hbm,
                    sidx, didx, rows_v, acc_sh, gsems, ssems):
    cid = lax.axis_index("c")
    sid = lax.axis_index("s")
    zero = jnp.zeros((16,), jnp.float32)

    def fill(i, _):
        for jj in range(DH // 16):
            rows_v[0, i, pl.ds(jj * 16, 16)] = zero
        return _

    lax.fori_loop(0, CH, fill, None)

    def zrow(b, _):
        pltpu.sync_copy(rows_v.at[0], acc_sh.at[pl.ds(sid * RPT + b * CH, CH)])
        return _

    lax.fori_loop(0, RPT // CH, zrow, None)
    plsc.subcore_barrier()

    def gather(j, b):
        pltpu.async_copy(y2_hbm.at[cid].at[sidx.at[j]], rows_v.at[b], gsems[b])

    def gwait(j, b):
        pltpu.make_async_copy(y2_hbm.at[cid].at[sidx.at[j]],
                              rows_v.at[b], gsems[b]).wait()

    def scatter(j, b):
        pltpu.async_copy(rows_v.at[b], acc_sh.at[didx.at[j]], ssems[b],
                         add=True)

    def swait(j, b):
        pltpu.make_async_copy(rows_v.at[b], acc_sh.at[didx.at[j]],
                              ssems[b]).wait()

    for p in range(KS // KH):
        pltpu.sync_copy(src_hbm.at[pl.ds(sid * KS + p * KH, KH)], sidx)
        pltpu.sync_copy(dst_hbm.at[pl.ds(sid * KS + p * KH, KH)], didx)

        for b in range(LOOKAHEAD):
            gather(b, b)

        def body(g, _):
            for u in range(NBUF):
                j = g * NBUF + u
                bi = (u + LOOKAHEAD) % NBUF

                @pl.when(j + LOOKAHEAD < KH)
                def _():
                    @pl.when(j + LOOKAHEAD >= NBUF)
                    def _():
                        swait(j + LOOKAHEAD - NBUF, bi)

                    gather(j + LOOKAHEAD, bi)

                gwait(j, u)
                scatter(j, u)
            return _

        lax.fori_loop(0, KH // NBUF, body, None)
        for u in range(NBUF):
            swait(KH - NBUF + u, u)
    plsc.subcore_barrier()
    pltpu.sync_copy(acc_sh.at[pl.ds(sid * RPT, RPT)],
                    out_hbm.at[cid, pl.ds(sid * RPT, RPT)])


def _prep_body(x_ref, ws_ref, wc_ref, b2_ref, degp_ref, xs_ref, y2_ref):
    xb = x_ref[...]
    dnums = (((1,), (1,)), ((), ()))
    xs = lax.dot_general(xb, ws_ref[...], dnums,
                         precision=lax.Precision.HIGHEST,
                         preferred_element_type=jnp.float32)
    xs_ref[...] = xs + b2_ref[...]
    xl = lax.dot_general(xb, wc_ref[...], dnums,
                         precision=lax.Precision.HIGHEST,
                         preferred_element_type=jnp.float32)
    deg = degp_ref[0, :, 0:1] + degp_ref[1, :, 0:1] + 1.0
    y = xl * lax.rsqrt(deg)
    y2_ref[0] = y[:, :DH]
    y2_ref[1] = y[:, DH:]


def _comb_body(xs_ref, y2_ref, accp_ref, degp_ref, o_ref):
    deg = degp_ref[0, :, 0:1] + degp_ref[1, :, 0:1] + 1.0
    dinv = lax.rsqrt(deg)
    s = accp_ref[...] + y2_ref[...]
    full = jnp.concatenate([s[0], s[1]], axis=1)
    o_ref[...] = xs_ref[...] + dinv * full


def kernel(x, edge_index, n1, n2, W_self, b_self, W_conv, b_conv):
    del n1, n2
    src = edge_index[0]
    dst = edge_index[1]
    pad = E_PAD - E
    src_p = jnp.concatenate(
        [src, jnp.full((pad,), N, jnp.int32)]).reshape(E_PAD // CH, CH)
    dst_p = jnp.concatenate(
        [dst, jnp.full((pad,), N, jnp.int32)]).reshape(E_PAD // CH, CH)
    x_pad = jnp.zeros((NP, D), jnp.float32).at[:N].set(x)
    b2 = (b_self + b_conv).reshape(1, D)

    degp = _deg_kernel(dst_p)

    blk = 2048
    xs, y2 = pl.pallas_call(
        _prep_body,
        grid=(NP // blk,),
        in_specs=[
            pl.BlockSpec((blk, D), lambda i: (i, 0)),
            pl.BlockSpec((D, D), lambda i: (0, 0)),
            pl.BlockSpec((D, D), lambda i: (0, 0)),
            pl.BlockSpec((1, D), lambda i: (0, 0)),
            pl.BlockSpec((NC, blk, DEGW), lambda i: (0, i, 0)),
        ],
        out_specs=[
            pl.BlockSpec((blk, D), lambda i: (i, 0)),
            pl.BlockSpec((NC, blk, DH), lambda i: (0, i, 0)),
        ],
        out_shape=[
            jax.ShapeDtypeStruct((NP, D), jnp.float32),
            jax.ShapeDtypeStruct((NC, NP, DH), jnp.float32),
        ],
    )(x_pad, W_self, W_conv, b2, degp)

    accp = _scatter_kernel(y2, src_p, dst_p)

    blk2 = 2000
    out = pl.pallas_call(
        _comb_body,
        grid=(N // blk2,),
        in_specs=[
            pl.BlockSpec((blk2, D), lambda i: (i, 0)),
            pl.BlockSpec((NC, blk2, DH), lambda i: (0, i, 0)),
            pl.BlockSpec((NC, blk2, DH), lambda i: (0, i, 0)),
            pl.BlockSpec((NC, blk2, DEGW), lambda i: (0, i, 0)),
        ],
        out_specs=pl.BlockSpec((blk2, D), lambda i: (i, 0)),
        out_shape=jax.ShapeDtypeStruct((N, D), jnp.float32),
    )(xs, y2, accp, degp)
    return out


# speedup vs baseline: 1.9124x; 1.9124x over previous
import functools

import jax
import jax.numpy as jnp
from jax import lax
from jax.experimental import pallas as pl
from jax.experimental.pallas import tpu as pltpu
from jax.experimental.pallas import tpu_sc as plsc

N = 10000
E = 320000
D = 128
DH = D // 2

NC = 2
NS = 16
NW = NC * NS

CH = 128
KD = 80
KS = 160
E_PAD = NW * KD * CH
NP = 10240
RPT = NP // NS
DEGW = 16
NBUF = 4
LOOKAHEAD = 2
KH = KS // 4
NPA = 10112
RPA = NPA // NS

_mesh = lambda: plsc.VectorSubcoreMesh(core_axis_name="c", subcore_axis_name="s")


@functools.partial(
    pl.kernel,
    out_type=jax.ShapeDtypeStruct((NC, NP, DEGW), jnp.float32),
    mesh=_mesh(),
    scratch_types=[
        pltpu.VMEM((KD, CH), jnp.int32),
        pltpu.VMEM((CH, DEGW), jnp.float32),
        pltpu.VMEM((CH, DEGW), jnp.float32),
        pltpu.VMEM_SHARED((NP, DEGW), jnp.float32),
    ],
    compiler_params=pltpu.CompilerParams(use_tc_tiling_on_sc=False),
)
def _deg_kernel(dst_hbm, out_hbm, idx_v, ones_v, zero_v, deg_sh):
    cid = lax.axis_index("c")
    sid = lax.axis_index("s")
    wid = cid * NS + sid
    one = jnp.full((16,), 1.0, jnp.float32)
    zero = jnp.zeros((16,), jnp.float32)

    def fill(i, _):
        ones_v[i, :] = one
        zero_v[i, :] = zero
        return _

    lax.fori_loop(0, CH, fill, None)

    def zrow(b, _):
        pltpu.sync_copy(zero_v, deg_sh.at[pl.ds(sid * RPT + b * CH, CH)])
        return _

    lax.fori_loop(0, RPT // CH, zrow, None)
    pltpu.sync_copy(dst_hbm.at[pl.ds(wid * KD, KD)], idx_v)
    plsc.subcore_barrier()

    def body(j, _):
        pltpu.sync_copy(ones_v, deg_sh.at[idx_v.at[j]], add=True)
        return _

    lax.fori_loop(0, KD, body, None)
    plsc.subcore_barrier()
    pltpu.sync_copy(deg_sh.at[pl.ds(sid * RPT, RPT)],
                    out_hbm.at[cid, pl.ds(sid * RPT, RPT)])


@functools.partial(
    pl.kernel,
    out_type=jax.ShapeDtypeStruct((NC, NPA, DH), jnp.float32),
    mesh=_mesh(),
    scratch_types=[
        pltpu.VMEM((KH, CH), jnp.int32),
        pltpu.VMEM((KH, CH), jnp.int32),
        pltpu.VMEM((NBUF, CH, DH), jnp.float32),
        pltpu.VMEM_SHARED((NPA, DH), jnp.float32),
        pltpu.VMEM_SHARED((NPA, DH), jnp.float32),
        [pltpu.SemaphoreType.DMA] * NBUF,
        [pltpu.SemaphoreType.DMA] * NBUF,
    ],
    compiler_params=pltpu.CompilerParams(use_tc_tiling_on_sc=False),
)
def _scatter_kernel(y2_hbm, src_hbm, dst_hbm, out_hbm,
                    sidx, didx, rows_v, y_sh, acc_sh, gsems, ssems):
    cid = lax.axis_index("c")
    sid = lax.axis_index("s")
    zero = jnp.zeros((16,), jnp.float32)

    pltpu.sync_copy(y2_hbm.at[cid, pl.ds(sid * RPA, RPA)],
                    y_sh.at[pl.ds(sid * RPA, RPA)])

    def fill(i, _):
        for jj in range(DH // 16):
            rows_v[0, i, pl.ds(jj * 16, 16)] = zero
        return _

    lax.fori_loop(0, CH, fill, None)
    for off in (0, 128, 256, 384, RPA - CH):
        pltpu.sync_copy(rows_v.at[0], acc_sh.at[pl.ds(sid * RPA + off, CH)])
    plsc.subcore_barrier()

    def gather(j, b):
        pltpu.async_copy(y_sh.at[sidx.at[j]], rows_v.at[b], gsems[b])

    def gwait(j, b):
        pltpu.make_async_copy(y_sh.at[sidx.at[j]],
                              rows_v.at[b], gsems[b]).wait()

    def scatter(j, b):
        pltpu.async_copy(rows_v.at[b], acc_sh.at[didx.at[j]], ssems[b],
                         add=True)

    def swait(j, b):
        pltpu.make_async_copy(rows_v.at[b], acc_sh.at[didx.at[j]],
                              ssems[b]).wait()

    for p in range(KS // KH):
        pltpu.sync_copy(src_hbm.at[pl.ds(sid * KS + p * KH, KH)], sidx)
        pltpu.sync_copy(dst_hbm.at[pl.ds(sid * KS + p * KH, KH)], didx)

        for b in range(LOOKAHEAD):
            gather(b, b)

        def body(g, _):
            for u in range(NBUF):
                j = g * NBUF + u
                bi = (u + LOOKAHEAD) % NBUF

                @pl.when(j + LOOKAHEAD < KH)
                def _():
                    @pl.when(j + LOOKAHEAD >= NBUF)
                    def _():
                        swait(j + LOOKAHEAD - NBUF, bi)

                    gather(j + LOOKAHEAD, bi)

                gwait(j, u)
                scatter(j, u)
            return _

        lax.fori_loop(0, KH // NBUF, body, None)
        for u in range(NBUF):
            swait(KH - NBUF + u, u)
    plsc.subcore_barrier()
    pltpu.sync_copy(acc_sh.at[pl.ds(sid * RPA, RPA)],
                    out_hbm.at[cid, pl.ds(sid * RPA, RPA)])


def _prep_body(x_ref, ws_ref, wc_ref, b2_ref, degp_ref, xs_ref, y2_ref):
    xb = x_ref[...]
    dnums = (((1,), (1,)), ((), ()))
    xs = lax.dot_general(xb, ws_ref[...], dnums,
                         precision=lax.Precision.HIGHEST,
                         preferred_element_type=jnp.float32)
    xs_ref[...] = xs + b2_ref[...]
    xl = lax.dot_general(xb, wc_ref[...], dnums,
                         precision=lax.Precision.HIGHEST,
                         preferred_element_type=jnp.float32)
    deg = degp_ref[0, :, 0:1] + degp_ref[1, :, 0:1] + 1.0
    y = xl * lax.rsqrt(deg)
    y2_ref[0] = y[:, :DH]
    y2_ref[1] = y[:, DH:]


def _comb_body(xs_ref, y2_ref, accp_ref, degp_ref, o_ref):
    deg = degp_ref[0, :, 0:1] + degp_ref[1, :, 0:1] + 1.0
    dinv = lax.rsqrt(deg)
    s = accp_ref[...] + y2_ref[...]
    full = jnp.concatenate([s[0], s[1]], axis=1)
    o_ref[...] = xs_ref[...] + dinv * full


def kernel(x, edge_index, n1, n2, W_self, b_self, W_conv, b_conv):
    del n1, n2
    src = edge_index[0]
    dst = edge_index[1]
    pad = E_PAD - E
    src_p = jnp.concatenate(
        [src, jnp.full((pad,), N, jnp.int32)]).reshape(E_PAD // CH, CH)
    dst_p = jnp.concatenate(
        [dst, jnp.full((pad,), N, jnp.int32)]).reshape(E_PAD // CH, CH)
    x_pad = jnp.zeros((NP, D), jnp.float32).at[:N].set(x)
    b2 = (b_self + b_conv).reshape(1, D)

    degp = _deg_kernel(dst_p)

    blk = 2048
    xs, y2 = pl.pallas_call(
        _prep_body,
        grid=(NP // blk,),
        in_specs=[
            pl.BlockSpec((blk, D), lambda i: (i, 0)),
            pl.BlockSpec((D, D), lambda i: (0, 0)),
            pl.BlockSpec((D, D), lambda i: (0, 0)),
            pl.BlockSpec((1, D), lambda i: (0, 0)),
            pl.BlockSpec((NC, blk, DEGW), lambda i: (0, i, 0)),
        ],
        out_specs=[
            pl.BlockSpec((blk, D), lambda i: (i, 0)),
            pl.BlockSpec((NC, blk, DH), lambda i: (0, i, 0)),
        ],
        out_shape=[
            jax.ShapeDtypeStruct((NP, D), jnp.float32),
            jax.ShapeDtypeStruct((NC, NP, DH), jnp.float32),
        ],
    )(x_pad, W_self, W_conv, b2, degp)

    accp = _scatter_kernel(y2, src_p, dst_p)

    blk2 = 2000
    out = pl.pallas_call(
        _comb_body,
        grid=(N // blk2,),
        in_specs=[
            pl.BlockSpec((blk2, D), lambda i: (i, 0)),
            pl.BlockSpec((NC, blk2, DH), lambda i: (0, i, 0)),
            pl.BlockSpec((NC, blk2, DH), lambda i: (0, i, 0)),
            pl.BlockSpec((NC, blk2, DEGW), lambda i: (0, i, 0)),
        ],
        out_specs=pl.BlockSpec((blk2, D), lambda i: (i, 0)),
        out_shape=jax.ShapeDtypeStruct((N, D), jnp.float32),
    )(xs, y2, accp, degp)
    return out


# speedup vs baseline: 1.9570x; 1.0233x over previous
import functools

import jax
import jax.numpy as jnp
from jax import lax
from jax.experimental import pallas as pl
from jax.experimental.pallas import tpu as pltpu
from jax.experimental.pallas import tpu_sc as plsc

N = 10000
E = 320000
D = 128
DH = D // 2

NC = 2
NS = 16
NW = NC * NS

CH = 128
KD = 80
KS = 160
E_PAD = NW * KD * CH
NP = 10240
RPT = NP // NS
DEGW = 16
NBUF = 4
LOOKAHEAD = 2
KH = KS // 4
NPA = 10112
RPA = NPA // NS

_mesh = lambda: plsc.VectorSubcoreMesh(core_axis_name="c", subcore_axis_name="s")


@functools.partial(
    pl.kernel,
    out_type=jax.ShapeDtypeStruct((NC, NP, DEGW), jnp.float32),
    mesh=_mesh(),
    scratch_types=[
        pltpu.VMEM((KD, CH), jnp.int32),
        pltpu.VMEM((CH, DEGW), jnp.float32),
        pltpu.VMEM((CH, DEGW), jnp.float32),
        pltpu.VMEM_SHARED((NP, DEGW), jnp.float32),
    ],
    compiler_params=pltpu.CompilerParams(use_tc_tiling_on_sc=False),
)
def _deg_kernel(dst_hbm, out_hbm, idx_v, ones_v, zero_v, deg_sh):
    cid = lax.axis_index("c")
    sid = lax.axis_index("s")
    wid = cid * NS + sid
    one = jnp.full((16,), 1.0, jnp.float32)
    zero = jnp.zeros((16,), jnp.float32)

    def fill(i, _):
        ones_v[i, :] = one
        zero_v[i, :] = zero
        return _

    lax.fori_loop(0, CH, fill, None)

    def zrow(b, _):
        pltpu.sync_copy(zero_v, deg_sh.at[pl.ds(sid * RPT + b * CH, CH)])
        return _

    lax.fori_loop(0, RPT // CH, zrow, None)
    pltpu.sync_copy(dst_hbm.at[pl.ds(wid * KD, KD)], idx_v)
    plsc.subcore_barrier()

    def body(j, _):
        pltpu.sync_copy(ones_v, deg_sh.at[idx_v.at[j]], add=True)
        return _

    lax.fori_loop(0, KD, body, None)
    plsc.subcore_barrier()
    pltpu.sync_copy(deg_sh.at[pl.ds(sid * RPT, RPT)],
                    out_hbm.at[cid, pl.ds(sid * RPT, RPT)])


@functools.partial(
    pl.kernel,
    out_type=jax.ShapeDtypeStruct((NC, NPA, DH), jnp.float32),
    mesh=_mesh(),
    scratch_types=[
        pltpu.VMEM((KH, CH), jnp.int32),
        pltpu.VMEM((KH, CH), jnp.int32),
        pltpu.VMEM((NBUF, CH, DH), jnp.float32),
        pltpu.VMEM_SHARED((NPA, DH), jnp.float32),
        pltpu.VMEM_SHARED((NPA, DH), jnp.float32),
        [pltpu.SemaphoreType.DMA] * NBUF,
        [pltpu.SemaphoreType.DMA] * NBUF,
    ],
    compiler_params=pltpu.CompilerParams(use_tc_tiling_on_sc=False),
)
def _scatter_kernel(y2_hbm, src_hbm, dst_hbm, out_hbm,
                    sidx, didx, rows_v, y_sh, acc_sh, gsems, ssems):
    cid = lax.axis_index("c")
    sid = lax.axis_index("s")
    zero = jnp.zeros((16,), jnp.float32)

    pltpu.sync_copy(y2_hbm.at[cid, pl.ds(sid * RPA, RPA)],
                    y_sh.at[pl.ds(sid * RPA, RPA)])

    def fill(i, _):
        for jj in range(DH // 16):
            rows_v[0, i, pl.ds(jj * 16, 16)] = zero
        return _

    lax.fori_loop(0, CH, fill, None)
    for off in (0, 128, 256, 384, RPA - CH):
        pltpu.sync_copy(rows_v.at[0], acc_sh.at[pl.ds(sid * RPA + off, CH)])
    plsc.subcore_barrier()

    def gather(j, b):
        pltpu.async_copy(y_sh.at[sidx.at[j]], rows_v.at[b], gsems[b])

    def gwait(j, b):
        pltpu.make_async_copy(y_sh.at[sidx.at[j]],
                              rows_v.at[b], gsems[b]).wait()

    def scatter(j, b):
        pltpu.async_copy(rows_v.at[b], acc_sh.at[didx.at[j]], ssems[b],
                         add=True)

    def swait(j, b):
        pltpu.make_async_copy(rows_v.at[b], acc_sh.at[didx.at[j]],
                              ssems[b]).wait()

    for p in range(KS // KH):
        pltpu.sync_copy(src_hbm.at[pl.ds(sid * KS + p * KH, KH)], sidx)
        pltpu.sync_copy(dst_hbm.at[pl.ds(sid * KS + p * KH, KH)], didx)

        for b in range(LOOKAHEAD):
            gather(b, b)

        def body(g, _):
            for u in range(NBUF):
                j = g * NBUF + u
                bi = (u + LOOKAHEAD) % NBUF

                @pl.when(j + LOOKAHEAD < KH)
                def _():
                    @pl.when(j + LOOKAHEAD >= NBUF)
                    def _():
                        swait(j + LOOKAHEAD - NBUF, bi)

                    gather(j + LOOKAHEAD, bi)

                gwait(j, u)
                scatter(j, u)
            return _

        lax.fori_loop(0, KH // NBUF, body, None)
        for u in range(NBUF):
            swait(KH - NBUF + u, u)
    plsc.subcore_barrier()
    pltpu.sync_copy(acc_sh.at[pl.ds(sid * RPA, RPA)],
                    out_hbm.at[cid, pl.ds(sid * RPA, RPA)])


def _mm_body(x_ref, ws_ref, wc_ref, b2_ref, xs_ref, xl_ref):
    xb = x_ref[...]
    dnums = (((1,), (1,)), ((), ()))
    xs = lax.dot_general(xb, ws_ref[...], dnums,
                         precision=lax.Precision.HIGHEST,
                         preferred_element_type=jnp.float32)
    xs_ref[...] = xs + b2_ref[...]
    xl_ref[...] = lax.dot_general(xb, wc_ref[...], dnums,
                                  precision=lax.Precision.HIGHEST,
                                  preferred_element_type=jnp.float32)


def _scale_body(xl_ref, degp_ref, y2_ref):
    deg = degp_ref[0, :, 0:1] + degp_ref[1, :, 0:1] + 1.0
    y = xl_ref[...] * lax.rsqrt(deg)
    y2_ref[0] = y[:, :DH]
    y2_ref[1] = y[:, DH:]


def _comb_body(xs_ref, y2_ref, accp_ref, degp_ref, o_ref):
    deg = degp_ref[0, :, 0:1] + degp_ref[1, :, 0:1] + 1.0
    dinv = lax.rsqrt(deg)
    s = accp_ref[...] + y2_ref[...]
    full = jnp.concatenate([s[0], s[1]], axis=1)
    o_ref[...] = xs_ref[...] + dinv * full


def kernel(x, edge_index, n1, n2, W_self, b_self, W_conv, b_conv):
    del n1, n2
    src = edge_index[0]
    dst = edge_index[1]
    pad = E_PAD - E
    src_p = jnp.concatenate(
        [src, jnp.full((pad,), N, jnp.int32)]).reshape(E_PAD // CH, CH)
    dst_p = jnp.concatenate(
        [dst, jnp.full((pad,), N, jnp.int32)]).reshape(E_PAD // CH, CH)
    b2 = (b_self + b_conv).reshape(1, D)

    blk = 2000
    xs, xl = pl.pallas_call(
        _mm_body,
        grid=(N // blk,),
        in_specs=[
            pl.BlockSpec((blk, D), lambda i: (i, 0)),
            pl.BlockSpec((D, D), lambda i: (0, 0)),
            pl.BlockSpec((D, D), lambda i: (0, 0)),
            pl.BlockSpec((1, D), lambda i: (0, 0)),
        ],
        out_specs=[
            pl.BlockSpec((blk, D), lambda i: (i, 0)),
            pl.BlockSpec((blk, D), lambda i: (i, 0)),
        ],
        out_shape=[
            jax.ShapeDtypeStruct((N, D), jnp.float32),
            jax.ShapeDtypeStruct((N, D), jnp.float32),
        ],
    )(x, W_self, W_conv, b2)

    degp = _deg_kernel(dst_p)

    y2 = pl.pallas_call(
        _scale_body,
        grid=(N // blk,),
        in_specs=[
            pl.BlockSpec((blk, D), lambda i: (i, 0)),
            pl.BlockSpec((NC, blk, DEGW), lambda i: (0, i, 0)),
        ],
        out_specs=pl.BlockSpec((NC, blk, DH), lambda i: (0, i, 0)),
        out_shape=jax.ShapeDtypeStruct((NC, NPA, DH), jnp.float32),
    )(xl, degp)

    accp = _scatter_kernel(y2, src_p, dst_p)

    blk2 = 2000
    out = pl.pallas_call(
        _comb_body,
        grid=(N // blk2,),
        in_specs=[
            pl.BlockSpec((blk2, D), lambda i: (i, 0)),
            pl.BlockSpec((NC, blk2, DH), lambda i: (0, i, 0)),
            pl.BlockSpec((NC, blk2, DH), lambda i: (0, i, 0)),
            pl.BlockSpec((NC, blk2, DEGW), lambda i: (0, i, 0)),
        ],
        out_specs=pl.BlockSpec((blk2, D), lambda i: (i, 0)),
        out_shape=jax.ShapeDtypeStruct((N, D), jnp.float32),
    )(xs, y2, accp, degp)
    return out
